# Initial kernel scaffold; baseline (speedup 1.0000x reference)
import jax
import jax.numpy as jnp
from jax.experimental import pallas as pl


def kernel(prediction, target, mask):
    raise NotImplementedError("write your pallas kernel here")


# speedup vs baseline: 12.1504x; 12.1504x over previous
import jax
import jax.numpy as jnp
import numpy as np
from jax.experimental import pallas as pl
from jax.experimental.pallas import tpu as pltpu

B, H, W = 16, 512, 512
NC = 16
ROWS = (B * H) // NC
NQP = 32
NRP = 32
NPASS = NQP + NRP

_M31 = 0x7FFFFFFF
_INT_MIN = np.int32(-2147483648)
_INT_MAX = np.int32(2147483647)


def _fmap(x):
    i = jax.lax.bitcast_convert_type(x, jnp.int32)
    return jnp.where(i >= 0, i, i ^ _M31)


def _finv_scalar(s):
    i = jnp.where(s >= 0, s, s ^ _M31)
    return jax.lax.bitcast_convert_type(i, jnp.float32)


def _mid(lo, hi):
    return (lo >> 1) + (hi >> 1) + (lo & hi & 1)


def _select_body(pred_ref, targ_ref, mask_ref, out_ref, st, fs):
    p = pl.program_id(0)
    c = pl.program_id(1)

    @pl.when((p == 0) & (c == 0))
    def _init():
        for j in range(4):
            st[j] = _INT_MIN
            st[4 + j] = _INT_MAX
            st[8 + j] = 0
        st[12] = 0
        st[13] = 0
        st[17] = 0
        st[18] = _INT_MAX
        st[19] = 0
        fs[4] = 0.0
        fs[5] = 0.0

    valid = mask_ref[...] != 0

    @pl.when(p < NQP)
    def _quant():
        sp = _fmap(pred_ref[...])
        stv = _fmap(targ_ref[...])

        @pl.when(p == 0)
        def _nv():
            st[13] += jnp.sum(valid.astype(jnp.int32))

        for j, s in ((0, sp), (1, sp), (2, stv), (3, stv)):
            piv = _mid(st[j], st[4 + j])
            st[8 + j] += jnp.sum(((s <= piv) & valid).astype(jnp.int32))

        @pl.when(c == NC - 1)
        def _update():
            nv = st[13]
            ranks = (nv // 100, (99 * nv) // 100, nv // 100, (99 * nv) // 100)
            for j in range(4):
                piv = _mid(st[j], st[4 + j])
                cond = st[8 + j] >= ranks[j] + 1
                new_hi = jnp.where(cond, piv, st[4 + j])
                new_lo = jnp.where(cond, st[j], piv + 1)
                st[j] = new_lo
                st[4 + j] = new_hi
                st[8 + j] = 0

            @pl.when(p == NQP - 1)
            def _fin_quant():
                fs[0] = _finv_scalar(st[0])
                fs[1] = _finv_scalar(st[1])
                fs[2] = _finv_scalar(st[2])
                fs[3] = _finv_scalar(st[3])

    @pl.when(p >= NQP)
    def _resid():
        invp = 1.0 / (fs[1] - fs[0] + 1e-8)
        invt = 1.0 / (fs[3] - fs[2] + 1e-8)
        pn = jnp.clip((pred_ref[...] - fs[0]) * invp, 0.0, 1.0)
        tn = jnp.clip((targ_ref[...] - fs[2]) * invt, 0.0, 1.0)
        res = jnp.abs(pn - tn)
        u = jax.lax.bitcast_convert_type(res, jnp.int32)
        piv = _mid(st[17], st[18])
        sel = (u <= piv) & valid
        st[12] += jnp.sum(sel.astype(jnp.int32))
        fs[4] += jnp.sum(jnp.where(sel, res, 0.0))

        @pl.when(c == NC - 1)
        def _update():
            k = (4 * st[13]) // 5
            piv2 = _mid(st[17], st[18])
            cond = st[12] >= k
            new_hi = jnp.where(cond, piv2, st[18])
            new_lo = jnp.where(cond, st[17], piv2 + 1)
            new_cb = jnp.where(cond, st[19], st[12])
            new_sb = jnp.where(cond, fs[5], fs[4])
            st[17] = new_lo
            st[18] = new_hi
            st[19] = new_cb
            fs[5] = new_sb
            st[12] = 0
            fs[4] = 0.0

            @pl.when(p == NPASS - 1)
            def _done():
                t = jax.lax.bitcast_convert_type(st[17], jnp.float32)
                nvf = st[13].astype(jnp.float32)
                kf = k.astype(jnp.float32)
                batch_loss = fs[5] + (kf - st[19].astype(jnp.float32)) * t
                out_ref[0] = fs[0]
                out_ref[1] = fs[1]
                out_ref[2] = fs[2]
                out_ref[3] = fs[3]
                out_ref[4] = batch_loss / (2.0 * nvf)
                out_ref[5] = nvf
                out_ref[6] = 0.0
                out_ref[7] = 0.0


_SCALES = (1, 2, 4, 8)


def _grad_body(scal_ref, pred_ref, targ_ref, mask_ref, out_ref, acc):
    i = pl.program_id(0)

    @pl.when(i == 0)
    def _init():
        for s in range(8):
            acc[s] = 0.0

    minp = scal_ref[0]
    maxp = scal_ref[1]
    mint = scal_ref[2]
    maxt = scal_ref[3]
    invp = 1.0 / (maxp - minp + 1e-8)
    invt = 1.0 / (maxt - mint + 1e-8)

    pn = jnp.clip((pred_ref[0] - minp) * invp, 0.0, 1.0)
    tn = jnp.clip((targ_ref[0] - mint) * invt, 0.0, 1.0)
    m = (mask_ref[0] != 0).astype(jnp.float32)
    d = m * (pn - tn)

    for si, s in enumerate(_SCALES):
        if s == 1:
            Ms = jnp.sum(m)
            dx = jnp.abs(d[:, s:] - d[:, :-s]) * (m[:, s:] * m[:, :-s])
            gx = jnp.sum(dx)
            dy = jnp.abs(d[s:, :] - d[:-s, :]) * (m[s:, :] * m[:-s, :])
            gy = jnp.sum(dy)
        else:
            rsel = (jax.lax.broadcasted_iota(jnp.int32, (H, W), 0) & (s - 1)) == 0
            csel = (jax.lax.broadcasted_iota(jnp.int32, (H, W), 1) & (s - 1)) == 0
            Ms = jnp.sum(jnp.where(rsel & csel, m, 0.0))
            dx = jnp.abs(d[:, s:] - d[:, :-s]) * (m[:, s:] * m[:, :-s])
            selx = rsel[:, : W - s] & csel[:, : W - s]
            gx = jnp.sum(jnp.where(selx, dx, 0.0))
            dy = jnp.abs(d[s:, :] - d[:-s, :]) * (m[s:, :] * m[:-s, :])
            sely = rsel[: H - s, :] & csel[: H - s, :]
            gy = jnp.sum(jnp.where(sely, dy, 0.0))
        il = gx + gy
        vb = Ms > 0.0
        acc[si] += jnp.where(vb, il, 0.0)
        acc[4 + si] += jnp.where(vb, Ms, 0.0)

    @pl.when(i == B - 1)
    def _fin():
        reg = 0.0
        for si in range(4):
            num = acc[si]
            den = acc[4 + si]
            reg += jnp.where(den > 0.0, num / jnp.maximum(den, 1e-8), 0.0)
        out_ref[0] = scal_ref[4] + 0.5 * reg


def _make_calls(interpret=False):
    select = pl.pallas_call(
        _select_body,
        grid=(NPASS, NC),
        in_specs=[
            pl.BlockSpec((ROWS, W), lambda p, c: (c, 0)),
            pl.BlockSpec((ROWS, W), lambda p, c: (c, 0)),
            pl.BlockSpec((ROWS, W), lambda p, c: (c, 0)),
        ],
        out_specs=pl.BlockSpec(memory_space=pltpu.SMEM),
        out_shape=jax.ShapeDtypeStruct((8,), jnp.float32),
        scratch_shapes=[
            pltpu.SMEM((24,), jnp.int32),
            pltpu.SMEM((8,), jnp.float32),
        ],
        interpret=interpret,
    )
    grad = pl.pallas_call(
        _grad_body,
        grid=(B,),
        in_specs=[
            pl.BlockSpec(memory_space=pltpu.SMEM),
            pl.BlockSpec((1, H, W), lambda i: (i, 0, 0)),
            pl.BlockSpec((1, H, W), lambda i: (i, 0, 0)),
            pl.BlockSpec((1, H, W), lambda i: (i, 0, 0)),
        ],
        out_specs=pl.BlockSpec(memory_space=pltpu.SMEM),
        out_shape=jax.ShapeDtypeStruct((1,), jnp.float32),
        scratch_shapes=[pltpu.SMEM((8,), jnp.float32)],
        interpret=interpret,
    )
    return select, grad


def _run(prediction, target, mask, interpret=False):
    select, grad = _make_calls(interpret)
    m8 = mask.astype(jnp.int8)
    p2 = prediction.reshape(B * H, W)
    t2 = target.reshape(B * H, W)
    m2 = m8.reshape(B * H, W)
    scal = select(p2, t2, m2)
    out = grad(scal, prediction, target, m8)
    return out[0]


def kernel(prediction, target, mask):
    return _run(prediction, target, mask, interpret=False)


# speedup vs baseline: 16.1554x; 1.3296x over previous
import jax
import jax.numpy as jnp
import numpy as np
from jax.experimental import pallas as pl
from jax.experimental.pallas import tpu as pltpu

B, H, W = 16, 512, 512
NC = 16
ROWS = (B * H) // NC
NQP = 32
NRP = 32

_M31 = 0x7FFFFFFF
_INT_MIN = np.int32(-2147483648)
_INT_MAX = np.int32(2147483647)
_FINITE_MAX = np.int32(0x7F7FFFFF)


def _fmap(x):
    i = jax.lax.bitcast_convert_type(x, jnp.int32)
    return jnp.where(i >= 0, i, i ^ _M31)


def _finv(s):
    i = jnp.where(s >= 0, s, s ^ _M31)
    return jax.lax.bitcast_convert_type(i, jnp.float32)


def _mid(lo, hi):
    return (lo >> 1) + (hi >> 1) + (lo & hi & 1)


def _key_body(pred_ref, targ_ref, mask_ref, spk_ref, stk_ref, nv_ref, acc):
    c = pl.program_id(0)

    @pl.when(c == 0)
    def _():
        acc[0] = 0

    valid = mask_ref[...] != 0
    spk_ref[...] = jnp.where(valid, _fmap(pred_ref[...]), _INT_MAX)
    stk_ref[...] = jnp.where(valid, _fmap(targ_ref[...]), _INT_MAX)
    acc[0] += jnp.sum(valid.astype(jnp.int32))

    @pl.when(c == NC - 1)
    def _():
        nv_ref[0] = acc[0]


def _qsearch_body(nv_ref, spk_ref, stk_ref, out_ref, st):
    p = pl.program_id(0)
    c = pl.program_id(1)

    @pl.when((p == 0) & (c == 0))
    def _init():
        for j in range(4):
            st[j] = _INT_MIN
            st[4 + j] = _INT_MAX
            st[8 + j] = 0

    sp = spk_ref[...]
    stv = stk_ref[...]
    for j, s in ((0, sp), (1, sp), (2, stv), (3, stv)):
        piv = _mid(st[j], st[4 + j])
        st[8 + j] += jnp.sum((s <= piv).astype(jnp.int32))

    @pl.when(c == NC - 1)
    def _update():
        nv = nv_ref[0]
        ranks = (nv // 100, (99 * nv) // 100, nv // 100, (99 * nv) // 100)
        for j in range(4):
            piv = _mid(st[j], st[4 + j])
            cond = st[8 + j] >= ranks[j] + 1
            new_hi = jnp.where(cond, piv, st[4 + j])
            new_lo = jnp.where(cond, st[j], piv + 1)
            st[j] = new_lo
            st[4 + j] = new_hi
            st[8 + j] = 0

        @pl.when(p == NQP - 1)
        def _fin():
            for j in range(4):
                out_ref[j] = _finv(st[j])


def _reskey_body(q_ref, spk_ref, stk_ref, resk_ref):
    invp = 1.0 / (q_ref[1] - q_ref[0] + 1e-8)
    invt = 1.0 / (q_ref[3] - q_ref[2] + 1e-8)
    spk = spk_ref[...]
    stk = stk_ref[...]
    valid = spk != _INT_MAX
    pn = jnp.clip((_finv(spk) - q_ref[0]) * invp, 0.0, 1.0)
    tn = jnp.clip((_finv(stk) - q_ref[2]) * invt, 0.0, 1.0)
    res = jnp.abs(pn - tn)
    u = jax.lax.bitcast_convert_type(res, jnp.int32)
    resk_ref[...] = jnp.where(valid, u, _INT_MAX)


def _rsearch_body(nv_ref, resk_ref, out_ref, st, fs):
    p = pl.program_id(0)
    c = pl.program_id(1)

    @pl.when((p == 0) & (c == 0))
    def _init():
        st[0] = 0
        st[1] = _FINITE_MAX
        st[2] = 0
        st[3] = 0
        fs[0] = 0.0
        fs[1] = 0.0

    u = resk_ref[...]
    piv = _mid(st[0], st[1])
    sel = u <= piv
    st[2] += jnp.sum(sel.astype(jnp.int32))
    fs[0] += jnp.sum(jnp.where(sel, jax.lax.bitcast_convert_type(u, jnp.float32), 0.0))

    @pl.when(c == NC - 1)
    def _update():
        k = (4 * nv_ref[0]) // 5
        piv2 = _mid(st[0], st[1])
        cond = st[2] >= k
        new_hi = jnp.where(cond, piv2, st[1])
        new_lo = jnp.where(cond, st[0], piv2 + 1)
        new_cb = jnp.where(cond, st[3], st[2])
        new_sb = jnp.where(cond, fs[1], fs[0])
        st[0] = new_lo
        st[1] = new_hi
        st[3] = new_cb
        fs[1] = new_sb
        st[2] = 0
        fs[0] = 0.0

        @pl.when(p == NRP - 1)
        def _done():
            t = jax.lax.bitcast_convert_type(st[0], jnp.float32)
            nvf = nv_ref[0].astype(jnp.float32)
            batch_loss = fs[1] + (k.astype(jnp.float32) - st[3].astype(jnp.float32)) * t
            out_ref[0] = batch_loss / (2.0 * nvf)


_SCALES = (1, 2, 4, 8)


def _grad_body(q_ref, l_ref, pred_ref, targ_ref, mask_ref, out_ref, acc):
    i = pl.program_id(0)

    @pl.when(i == 0)
    def _init():
        for s in range(8):
            acc[s] = 0.0

    minp = q_ref[0]
    mint = q_ref[2]
    invp = 1.0 / (q_ref[1] - minp + 1e-8)
    invt = 1.0 / (q_ref[3] - mint + 1e-8)

    pn = jnp.clip((pred_ref[0] - minp) * invp, 0.0, 1.0)
    tn = jnp.clip((targ_ref[0] - mint) * invt, 0.0, 1.0)
    m = (mask_ref[0] != 0).astype(jnp.float32)
    d = m * (pn - tn)

    for si, s in enumerate(_SCALES):
        if s == 1:
            Ms = jnp.sum(m)
            gx = jnp.sum(jnp.abs(d[:, s:] - d[:, :-s]) * (m[:, s:] * m[:, :-s]))
            gy = jnp.sum(jnp.abs(d[s:, :] - d[:-s, :]) * (m[s:, :] * m[:-s, :]))
        else:
            rsel = (jax.lax.broadcasted_iota(jnp.int32, (H, W), 0) & (s - 1)) == 0
            csel = (jax.lax.broadcasted_iota(jnp.int32, (H, W), 1) & (s - 1)) == 0
            Ms = jnp.sum(jnp.where(rsel & csel, m, 0.0))
            dx = jnp.abs(d[:, s:] - d[:, :-s]) * (m[:, s:] * m[:, :-s])
            gx = jnp.sum(jnp.where(rsel[:, : W - s] & csel[:, : W - s], dx, 0.0))
            dy = jnp.abs(d[s:, :] - d[:-s, :]) * (m[s:, :] * m[:-s, :])
            gy = jnp.sum(jnp.where(rsel[: H - s, :] & csel[: H - s, :], dy, 0.0))
        vb = Ms > 0.0
        acc[si] += jnp.where(vb, gx + gy, 0.0)
        acc[4 + si] += jnp.where(vb, Ms, 0.0)

    @pl.when(i == B - 1)
    def _fin():
        reg = 0.0
        for si in range(4):
            reg += jnp.where(acc[4 + si] > 0.0,
                             acc[si] / jnp.maximum(acc[4 + si], 1e-8), 0.0)
        out_ref[0] = l_ref[0] + 0.5 * reg


def _chunk_spec():
    return pl.BlockSpec((ROWS, W), lambda *g: (g[-1], 0))


def _make_calls(interpret=False):
    keyp = pl.pallas_call(
        _key_body,
        grid=(NC,),
        in_specs=[_chunk_spec(), _chunk_spec(), _chunk_spec()],
        out_specs=[
            _chunk_spec(),
            _chunk_spec(),
            pl.BlockSpec(memory_space=pltpu.SMEM),
        ],
        out_shape=[
            jax.ShapeDtypeStruct((B * H, W), jnp.int32),
            jax.ShapeDtypeStruct((B * H, W), jnp.int32),
            jax.ShapeDtypeStruct((1,), jnp.int32),
        ],
        scratch_shapes=[pltpu.SMEM((1,), jnp.int32)],
        interpret=interpret,
    )
    qsearch = pl.pallas_call(
        _qsearch_body,
        grid=(NQP, NC),
        in_specs=[
            pl.BlockSpec(memory_space=pltpu.SMEM),
            _chunk_spec(),
            _chunk_spec(),
        ],
        out_specs=pl.BlockSpec(memory_space=pltpu.SMEM),
        out_shape=jax.ShapeDtypeStruct((4,), jnp.float32),
        scratch_shapes=[pltpu.SMEM((12,), jnp.int32)],
        interpret=interpret,
    )
    reskey = pl.pallas_call(
        _reskey_body,
        grid=(NC,),
        in_specs=[
            pl.BlockSpec(memory_space=pltpu.SMEM),
            _chunk_spec(),
            _chunk_spec(),
        ],
        out_specs=_chunk_spec(),
        out_shape=jax.ShapeDtypeStruct((B * H, W), jnp.int32),
        interpret=interpret,
    )
    rsearch = pl.pallas_call(
        _rsearch_body,
        grid=(NRP, NC),
        in_specs=[
            pl.BlockSpec(memory_space=pltpu.SMEM),
            _chunk_spec(),
        ],
        out_specs=pl.BlockSpec(memory_space=pltpu.SMEM),
        out_shape=jax.ShapeDtypeStruct((1,), jnp.float32),
        scratch_shapes=[
            pltpu.SMEM((4,), jnp.int32),
            pltpu.SMEM((2,), jnp.float32),
        ],
        interpret=interpret,
    )
    grad = pl.pallas_call(
        _grad_body,
        grid=(B,),
        in_specs=[
            pl.BlockSpec(memory_space=pltpu.SMEM),
            pl.BlockSpec(memory_space=pltpu.SMEM),
            pl.BlockSpec((1, H, W), lambda i: (i, 0, 0)),
            pl.BlockSpec((1, H, W), lambda i: (i, 0, 0)),
            pl.BlockSpec((1, H, W), lambda i: (i, 0, 0)),
        ],
        out_specs=pl.BlockSpec(memory_space=pltpu.SMEM),
        out_shape=jax.ShapeDtypeStruct((1,), jnp.float32),
        scratch_shapes=[pltpu.SMEM((8,), jnp.float32)],
        interpret=interpret,
    )
    return keyp, qsearch, reskey, rsearch, grad


def _run(prediction, target, mask, interpret=False):
    keyp, qsearch, reskey, rsearch, grad = _make_calls(interpret)
    m8 = mask.astype(jnp.int8)
    p2 = prediction.reshape(B * H, W)
    t2 = target.reshape(B * H, W)
    m2 = m8.reshape(B * H, W)
    spk, stk, nv = keyp(p2, t2, m2)
    quant = qsearch(nv, spk, stk)
    resk = reskey(quant, spk, stk)
    loss1 = rsearch(nv, resk)
    out = grad(quant, loss1, prediction, target, m8)
    return out[0]


def kernel(prediction, target, mask):
    return _run(prediction, target, mask, interpret=False)
